# baseline (speedup 1.0000x reference)
import functools

import jax
import jax.numpy as jnp
from jax import lax
from jax.experimental import pallas as pl
from jax.experimental.pallas import tpu as pltpu
from jax.experimental.pallas import tpu_sc as plsc

_VOC = 1000000
_ROWS = 4096
_COLS = 200
_D = 64
_NC = 2
_NS = 16
_NW = _NC * _NS
_BB = _ROWS // 128


def _gather_body(xT_hbm, W_hbm, out_hbm, idx_v, g0, g1, t0, t1, sg0, sg1, sw0, sw1):
    w = lax.axis_index("s") * _NC + lax.axis_index("c")
    gbufs = (g0, g1)
    tbufs = (t0, t1)
    sgs = (sg0, sg1)
    sws = (sw0, sw1)

    pltpu.sync_copy(xT_hbm.at[:, pl.ds(w * 128, 128)], idx_v)

    def start_gather(p, t):
        pltpu.async_copy(W_hbm.at[idx_v.at[t]], gbufs[p], sgs[p])

    def wait_gather(p):
        pltpu.make_async_copy(W_hbm.at[pl.ds(0, 128)], gbufs[p], sgs[p]).wait()

    def start_wb(p, t):
        pltpu.async_copy(tbufs[p], out_hbm.at[t, :, w, :, :], sws[p])

    def wait_wb(p):
        pltpu.make_async_copy(
            tbufs[p], out_hbm.at[0, :, w, :, :], sws[p]
        ).wait()

    lanes = lax.iota(jnp.int32, 16)

    def transpose(p):
        g, tb = gbufs[p], tbufs[p]

        def grp(j, carry):
            rows = j * 16 + lanes
            for d in range(_D):
                col = jnp.full((16,), d, jnp.int32)
                val = plsc.load_gather(g, [rows, col])
                tb[d // 8, d % 8, pl.ds(j * 16, 16)] = val
            return carry

        lax.fori_loop(0, 8, grp, 0)

    start_gather(0, 0)
    start_gather(1, 1)

    for p in range(2):
        wait_gather(p)
        transpose(p)
        start_wb(p, p)
        start_gather(p, p + 2)

    def loop_body(i, carry):
        for p in range(2):
            t = 2 * i + p
            wait_gather(p)
            wait_wb(p)
            transpose(p)
            start_wb(p, t)

            @pl.when(t + 2 < _COLS)
            def _():
                start_gather(p, t + 2)

        return carry

    lax.fori_loop(1, _COLS // 2, loop_body, 0)

    for p in range(2):
        wait_wb(p)


_gather = functools.partial(
    pl.kernel,
    out_type=jax.ShapeDtypeStruct((_COLS, 8, _BB, 8, 128), jnp.float32),
    mesh=plsc.VectorSubcoreMesh(core_axis_name="c", subcore_axis_name="s"),
    scratch_types=[
        pltpu.VMEM((_COLS, 128), jnp.int32),
        pltpu.VMEM((128, _D), jnp.float32),
        pltpu.VMEM((128, _D), jnp.float32),
        pltpu.VMEM((8, 8, 128), jnp.float32),
        pltpu.VMEM((8, 8, 128), jnp.float32),
        pltpu.SemaphoreType.DMA,
        pltpu.SemaphoreType.DMA,
        pltpu.SemaphoreType.DMA,
        pltpu.SemaphoreType.DMA,
    ],
    compiler_params=pltpu.CompilerParams(
        use_tc_tiling_on_sc=False, needs_layout_passes=False
    ),
)(_gather_body)


def _mask_body(x_ref, m_ref):
    m_ref[...] = x_ref[...] != 0


_mask = pl.pallas_call(
    _mask_body,
    out_shape=jax.ShapeDtypeStruct((_ROWS, _COLS), jnp.bool_),
)


def kernel(x, W):
    out5d = _gather(x.T, W)
    out = out5d.transpose(2, 4, 0, 1, 3).reshape(_ROWS, _COLS, _D)
    pad_mask = _mask(x)
    return out, pad_mask


# speedup vs baseline: 1.4659x; 1.4659x over previous
import functools

import jax
import jax.numpy as jnp
from jax import lax
from jax.experimental import pallas as pl
from jax.experimental.pallas import tpu as pltpu
from jax.experimental.pallas import tpu_sc as plsc

_VOC = 1000000
_ROWS = 4096
_COLS = 200
_D = 64
_B = _ROWS * _COLS
_NC = 2
_NS = 16
_NW = _NC * _NS
_BPW = _B // _NW
_C = 800
_NCHUNK = _BPW // _C

_DP_R = 2000
_DP_G = _VOC // _DP_R


def _gather_body(x_hbm, W_hbm, out_hbm, idx_v, buf0, buf1, sg0, sg1, sw0, sw1):
    wid = lax.axis_index("s") * _NC + lax.axis_index("c")
    base = wid * _BPW
    bufs = (buf0, buf1)
    sgs = (sg0, sg1)
    sws = (sw0, sw1)

    pltpu.sync_copy(x_hbm.at[pl.ds(base, _BPW)], idx_v)

    def start_gather(b, k):
        pltpu.async_copy(W_hbm.at[idx_v.at[pl.ds(k * _C, _C)]], bufs[b], sgs[b])

    def start_wb(b, k):
        pltpu.async_copy(
            bufs[b], out_hbm.at[pl.ds(base + k * _C, _C), pl.ds(0, _D)], sws[b]
        )

    def wait_gather(b):
        pltpu.make_async_copy(W_hbm.at[pl.ds(0, _C)], bufs[b], sgs[b]).wait()

    def wait_wb(b):
        pltpu.make_async_copy(
            bufs[b], out_hbm.at[pl.ds(base, _C), pl.ds(0, _D)], sws[b]
        ).wait()

    for b in range(2):
        start_gather(b, b)

    def step(i, carry):
        for b in range(2):
            k = 2 * i + b
            wait_gather(b)
            start_wb(b, k)
        for b in range(2):
            wait_wb(b)
            start_gather(b, 2 * i + 2 + b)
        return carry

    lax.fori_loop(0, _NCHUNK // 2 - 1, step, 0)

    for b in range(2):
        k = _NCHUNK - 2 + b
        wait_gather(b)
        start_wb(b, k)
    for b in range(2):
        wait_wb(b)


_gather = functools.partial(
    pl.kernel,
    out_type=jax.ShapeDtypeStruct((_B, 128), jnp.float32),
    mesh=plsc.VectorSubcoreMesh(core_axis_name="c", subcore_axis_name="s"),
    scratch_types=[
        pltpu.VMEM((_BPW,), jnp.int32),
        pltpu.VMEM((_C, _D), jnp.float32),
        pltpu.VMEM((_C, _D), jnp.float32),
        pltpu.SemaphoreType.DMA,
        pltpu.SemaphoreType.DMA,
        pltpu.SemaphoreType.DMA,
        pltpu.SemaphoreType.DMA,
    ],
    compiler_params=pltpu.CompilerParams(use_tc_tiling_on_sc=False),
)(_gather_body)


def _depad_body(w_ref, o_ref):
    w = w_ref[...].reshape(_DP_R // 2, 2, _D)
    o_ref[...] = jnp.concatenate([w[:, 0, :], w[:, 1, :]], axis=1)


_depad = pl.pallas_call(
    _depad_body,
    grid=(_DP_G,),
    in_specs=[pl.BlockSpec((_DP_R, _D), lambda i: (i, 0))],
    out_specs=pl.BlockSpec((_DP_R // 2, 128), lambda i: (i, 0)),
    out_shape=jax.ShapeDtypeStruct((_VOC // 2, 128), jnp.float32),
    compiler_params=pltpu.CompilerParams(
        dimension_semantics=("arbitrary",),
    ),
)


def _mask_body(x_ref, m_ref):
    m_ref[...] = x_ref[...] != 0


_mask = pl.pallas_call(
    _mask_body,
    out_shape=jax.ShapeDtypeStruct((_ROWS, _COLS), jnp.bool_),
)


def kernel(x, W):
    W_lin = _depad(W).reshape(_VOC, _D)
    out = _gather(x.reshape(_B), W_lin)
    pad_mask = _mask(x)
    return out.reshape(_ROWS, _COLS, 128)[..., :_D], pad_mask


# speedup vs baseline: 1.9405x; 1.3238x over previous
import functools

import jax
import jax.numpy as jnp
from jax import lax
from jax.experimental import pallas as pl
from jax.experimental.pallas import tpu as pltpu
from jax.experimental.pallas import tpu_sc as plsc

_VOC = 1000000
_ROWS = 4096
_COLS = 200
_D = 64
_B = _ROWS * _COLS
_NC = 2
_NS = 16
_NW = _NC * _NS
_BPW = _B // _NW
_C = 800
_NCHUNK = _BPW // _C


def _gather_body(x_hbm, W_hbm, out_hbm, idx_v, buf0, buf1, sg0, sg1, sw0, sw1):
    wid = lax.axis_index("s") * _NC + lax.axis_index("c")
    base = wid * _BPW
    bufs = (buf0, buf1)
    sgs = (sg0, sg1)
    sws = (sw0, sw1)

    pltpu.sync_copy(x_hbm.at[pl.ds(base, _BPW)], idx_v)

    def start_gather(b, k):
        pltpu.async_copy(W_hbm.at[idx_v.at[pl.ds(k * _C, _C)]], bufs[b], sgs[b])

    def start_wb(b, k):
        pltpu.async_copy(
            bufs[b], out_hbm.at[pl.ds(base + k * _C, _C), pl.ds(0, _D)], sws[b]
        )

    def wait_gather(b):
        pltpu.make_async_copy(W_hbm.at[pl.ds(0, _C)], bufs[b], sgs[b]).wait()

    def wait_wb(b):
        pltpu.make_async_copy(
            bufs[b], out_hbm.at[pl.ds(base, _C), pl.ds(0, _D)], sws[b]
        ).wait()

    for b in range(2):
        start_gather(b, b)

    def step(i, carry):
        for b in range(2):
            k = 2 * i + b
            wait_gather(b)
            start_wb(b, k)
        for b in range(2):
            wait_wb(b)
            start_gather(b, 2 * i + 2 + b)
        return carry

    lax.fori_loop(0, _NCHUNK // 2 - 1, step, 0)

    for b in range(2):
        k = _NCHUNK - 2 + b
        wait_gather(b)
        start_wb(b, k)
    for b in range(2):
        wait_wb(b)


_gather = functools.partial(
    pl.kernel,
    out_type=jax.ShapeDtypeStruct((_B, 128), jnp.float32),
    mesh=plsc.VectorSubcoreMesh(core_axis_name="c", subcore_axis_name="s"),
    scratch_types=[
        pltpu.VMEM((_BPW,), jnp.int32),
        pltpu.VMEM((_C, _D), jnp.float32),
        pltpu.VMEM((_C, _D), jnp.float32),
        pltpu.SemaphoreType.DMA,
        pltpu.SemaphoreType.DMA,
        pltpu.SemaphoreType.DMA,
        pltpu.SemaphoreType.DMA,
    ],
    compiler_params=pltpu.CompilerParams(use_tc_tiling_on_sc=False),
)(_gather_body)


def _mask_body(x_ref, m_ref):
    m_ref[...] = x_ref[...] != 0


_mask = pl.pallas_call(
    _mask_body,
    out_shape=jax.ShapeDtypeStruct((_ROWS, _COLS), jnp.bool_),
)


def kernel(x, W):
    out = _gather(x.reshape(_B), W)
    pad_mask = _mask(x)
    return out.reshape(_ROWS, _COLS, 128)[..., :_D], pad_mask


# speedup vs baseline: 1.9496x; 1.0047x over previous
import functools

import jax
import jax.numpy as jnp
from jax import lax
from jax.experimental import pallas as pl
from jax.experimental.pallas import tpu as pltpu
from jax.experimental.pallas import tpu_sc as plsc

_VOC = 1000000
_ROWS = 4096
_COLS = 200
_D = 64
_B = _ROWS * _COLS
_NC = 2
_NS = 16
_NW = _NC * _NS
_BPW = _B // _NW
_C = 400
_NB = 4
_NCHUNK = _BPW // _C


def _gather_body(
    x_hbm, W_hbm, out_hbm, idx_v,
    buf0, buf1, buf2, buf3,
    sg0, sg1, sg2, sg3, sw0, sw1, sw2, sw3,
):
    wid = lax.axis_index("s") * _NC + lax.axis_index("c")
    base = wid * _BPW
    bufs = (buf0, buf1, buf2, buf3)
    sgs = (sg0, sg1, sg2, sg3)
    sws = (sw0, sw1, sw2, sw3)

    pltpu.sync_copy(x_hbm.at[pl.ds(base, _BPW)], idx_v)

    def start_gather(b, k):
        pltpu.async_copy(W_hbm.at[idx_v.at[pl.ds(k * _C, _C)]], bufs[b], sgs[b])

    def start_wb(b, k):
        pltpu.async_copy(
            bufs[b], out_hbm.at[pl.ds(base + k * _C, _C), pl.ds(0, _D)], sws[b]
        )

    def wait_gather(b):
        pltpu.make_async_copy(W_hbm.at[pl.ds(0, _C)], bufs[b], sgs[b]).wait()

    def wait_wb(b):
        pltpu.make_async_copy(
            bufs[b], out_hbm.at[pl.ds(base, _C), pl.ds(0, _D)], sws[b]
        ).wait()

    for b in range(_NB):
        start_gather(b, b)

    def step(i, carry):
        for b in range(_NB):
            k = _NB * i + b
            wait_gather(b)
            start_wb(b, k)
        for b in range(_NB):
            wait_wb(b)
            start_gather(b, _NB * i + _NB + b)
        return carry

    lax.fori_loop(0, _NCHUNK // _NB - 1, step, 0)

    for b in range(_NB):
        k = _NCHUNK - _NB + b
        wait_gather(b)
        start_wb(b, k)
    for b in range(_NB):
        wait_wb(b)


_gather = functools.partial(
    pl.kernel,
    out_type=jax.ShapeDtypeStruct((_B, 128), jnp.float32),
    mesh=plsc.VectorSubcoreMesh(core_axis_name="c", subcore_axis_name="s"),
    scratch_types=[
        pltpu.VMEM((_BPW,), jnp.int32),
        pltpu.VMEM((_C, _D), jnp.float32),
        pltpu.VMEM((_C, _D), jnp.float32),
        pltpu.VMEM((_C, _D), jnp.float32),
        pltpu.VMEM((_C, _D), jnp.float32),
        pltpu.SemaphoreType.DMA,
        pltpu.SemaphoreType.DMA,
        pltpu.SemaphoreType.DMA,
        pltpu.SemaphoreType.DMA,
        pltpu.SemaphoreType.DMA,
        pltpu.SemaphoreType.DMA,
        pltpu.SemaphoreType.DMA,
        pltpu.SemaphoreType.DMA,
    ],
    compiler_params=pltpu.CompilerParams(use_tc_tiling_on_sc=False),
)(_gather_body)


def _mask_body(x_ref, m_ref):
    m_ref[...] = x_ref[...] != 0


_mask = pl.pallas_call(
    _mask_body,
    out_shape=jax.ShapeDtypeStruct((_ROWS, _COLS), jnp.bool_),
)


def kernel(x, W):
    out = _gather(x.reshape(_B), W)
    pad_mask = _mask(x)
    return out.reshape(_ROWS, _COLS, 128)[..., :_D], pad_mask
